# baseline (speedup 1.0000x reference)
import functools

import jax
import jax.numpy as jnp
from jax import lax
from jax.experimental import pallas as pl
from jax.experimental.pallas import tpu as pltpu
from jax.experimental.pallas import tpu_sc as plsc

VOCAB_ROWS = 100000
DIM = 64
WORDS = VOCAB_ROWS * DIM

_NUM_CORES = 2
_NUM_SUBCORES = 16
_NUM_WORKERS = _NUM_CORES * _NUM_SUBCORES
_WWORDS = WORDS // _NUM_WORKERS
_CWORDS = 50000


@functools.partial(
    pl.kernel,
    mesh=plsc.VectorSubcoreMesh(core_axis_name="c", subcore_axis_name="s"),
    out_type=jax.ShapeDtypeStruct((WORDS,), jnp.float32),
    scratch_types=[
        pltpu.VMEM((_CWORDS,), jnp.float32),
        pltpu.VMEM((_CWORDS,), jnp.float32),
        pltpu.SemaphoreType.DMA,
        pltpu.SemaphoreType.DMA,
        pltpu.SemaphoreType.DMA,
        pltpu.SemaphoreType.DMA,
    ],
)
def _copy_kernel(in_hbm, out_hbm, buf0, buf1, si0, si1, so0, so1):
    wid = lax.axis_index("s") * _NUM_CORES + lax.axis_index("c")
    base = pl.multiple_of(wid * _WWORDS, 8)
    bufs = (buf0, buf1)
    sin = (si0, si1)
    sout = (so0, so1)
    n = _WWORDS // _CWORDS

    h_in = [None, None]
    h_out = [None, None]
    h_in[0] = pltpu.async_copy(
        in_hbm.at[pl.ds(base, _CWORDS)], bufs[0], sin[0])
    for i in range(n):
        b = i % 2
        h_in[b].wait()
        if i + 1 < n:
            if h_out[1 - b] is not None:
                h_out[1 - b].wait()
            h_in[1 - b] = pltpu.async_copy(
                in_hbm.at[pl.ds(base + (i + 1) * _CWORDS, _CWORDS)],
                bufs[1 - b], sin[1 - b])
        h_out[b] = pltpu.async_copy(
            bufs[b], out_hbm.at[pl.ds(base + i * _CWORDS, _CWORDS)], sout[b])
    for h in h_out:
        if h is not None:
            h.wait()


def kernel(embeddings):
    return _copy_kernel(embeddings.reshape(WORDS)).reshape(VOCAB_ROWS, DIM)


# speedup vs baseline: 1.3105x; 1.3105x over previous
import functools

import jax
import jax.numpy as jnp
from jax import lax
from jax.experimental import pallas as pl
from jax.experimental.pallas import tpu as pltpu
from jax.experimental.pallas import tpu_sc as plsc

VOCAB_ROWS = 100000
DIM = 64

_NUM_CORES = 2
_NUM_SUBCORES = 16
_NUM_WORKERS = _NUM_CORES * _NUM_SUBCORES
_WCHUNK = 3128
_LAST = VOCAB_ROWS - (_NUM_WORKERS - 1) * _WCHUNK
_CROWS = 504


def _chunk_sizes(total):
    sizes = []
    while total > 0:
        sizes.append(min(_CROWS, total))
        total -= sizes[-1]
    return sizes


@functools.partial(
    pl.kernel,
    mesh=plsc.VectorSubcoreMesh(core_axis_name="c", subcore_axis_name="s"),
    out_type=jax.ShapeDtypeStruct((VOCAB_ROWS, DIM), jnp.float32),
    compiler_params=pltpu.CompilerParams(use_tc_tiling_on_sc=True),
    scratch_types=[
        pltpu.VMEM((_CROWS, DIM), jnp.float32),
        pltpu.VMEM((_CROWS, DIM), jnp.float32),
        pltpu.SemaphoreType.DMA,
        pltpu.SemaphoreType.DMA,
        pltpu.SemaphoreType.DMA,
        pltpu.SemaphoreType.DMA,
    ],
)
def _copy_kernel(in_hbm, out_hbm, buf0, buf1, si0, si1, so0, so1):
    wid = lax.axis_index("s") * _NUM_CORES + lax.axis_index("c")
    base = pl.multiple_of(wid * _WCHUNK, 8)
    bufs = (buf0, buf1)
    sin = (si0, si1)
    sout = (so0, so1)

    def copy_range(start, total):
        sizes = _chunk_sizes(total)
        n = len(sizes)
        h_in = [None, None]
        h_out = [None, None]
        offs = []
        off = 0
        for sz in sizes:
            offs.append(off)
            off += sz
        h_in[0] = pltpu.async_copy(
            in_hbm.at[pl.ds(start + offs[0], sizes[0])],
            bufs[0].at[pl.ds(0, sizes[0])], sin[0])
        for i in range(n):
            b = i % 2
            h_in[b].wait()
            if i + 1 < n:
                if h_out[1 - b] is not None:
                    h_out[1 - b].wait()
                h_in[1 - b] = pltpu.async_copy(
                    in_hbm.at[pl.ds(start + offs[i + 1], sizes[i + 1])],
                    bufs[1 - b].at[pl.ds(0, sizes[i + 1])], sin[1 - b])
            h_out[b] = pltpu.async_copy(
                bufs[b].at[pl.ds(0, sizes[i])],
                out_hbm.at[pl.ds(start + offs[i], sizes[i])], sout[b])
        for h in h_out:
            if h is not None:
                h.wait()

    @pl.when(wid < _NUM_WORKERS - 1)
    def _():
        copy_range(base, _WCHUNK)

    @pl.when(wid == _NUM_WORKERS - 1)
    def _():
        copy_range((_NUM_WORKERS - 1) * _WCHUNK, _LAST)


def kernel(embeddings):
    return _copy_kernel(embeddings)


# speedup vs baseline: 1.3441x; 1.0256x over previous
import jax
import jax.numpy as jnp
from jax.experimental import pallas as pl

VOCAB_ROWS = 100000
DIM = 64
_BLK = 2000
_GRID = VOCAB_ROWS // _BLK


def _copy_body(i_ref, o_ref):
    o_ref[...] = i_ref[...]


def kernel(embeddings):
    return pl.pallas_call(
        _copy_body,
        grid=(_GRID,),
        in_specs=[pl.BlockSpec((_BLK, DIM), lambda i: (i, 0))],
        out_specs=pl.BlockSpec((_BLK, DIM), lambda i: (i, 0)),
        out_shape=jax.ShapeDtypeStruct((VOCAB_ROWS, DIM), jnp.float32),
    )(embeddings)


# speedup vs baseline: 1.5726x; 1.1700x over previous
import jax
import jax.numpy as jnp
from jax.experimental import pallas as pl

VOCAB_ROWS = 100000
DIM = 64
_BLK = 10000
_GRID = VOCAB_ROWS // _BLK


def _copy_body(i_ref, o_ref):
    o_ref[...] = i_ref[...]


def kernel(embeddings):
    return pl.pallas_call(
        _copy_body,
        grid=(_GRID,),
        in_specs=[pl.BlockSpec((_BLK, DIM), lambda i: (i, 0))],
        out_specs=pl.BlockSpec((_BLK, DIM), lambda i: (i, 0)),
        out_shape=jax.ShapeDtypeStruct((VOCAB_ROWS, DIM), jnp.float32),
    )(embeddings)
